# baseline (speedup 1.0000x reference)
import functools

import jax
import jax.numpy as jnp
from jax import lax
from jax.experimental import pallas as pl
from jax.experimental.pallas import tpu as pltpu
from jax.experimental.pallas import tpu_sc as plsc

NC = 2
NS = 16
L = 16
NW = NC * NS

N = 10000
NPAD = 10240
D = 128
E = 320000
EPT = E // NW
CH = 80
NCHUNK = EPT // CH
RPT = NPAD // NS
LRPT = N - (NS - 1) * RPT
DG = D // L

_MESH = plsc.VectorSubcoreMesh(
    core_axis_name="c", subcore_axis_name="s", num_cores=NC, num_subcores=NS)


def _wid():
    return lax.axis_index("c") * NS + lax.axis_index("s")


def _writeback(acc, out_hbm, c, s):
    @pl.when(s < NS - 1)
    def _():
        pltpu.sync_copy(acc.at[pl.ds(s * RPT, RPT)],
                        out_hbm.at[pl.ds(c * N + s * RPT, RPT)])

    @pl.when(s == NS - 1)
    def _():
        pltpu.sync_copy(acc.at[pl.ds((NS - 1) * RPT, LRPT)],
                        out_hbm.at[pl.ds(c * N + (NS - 1) * RPT, LRPT)])


@functools.partial(
    pl.kernel,
    out_type=jax.ShapeDtypeStruct((NC * N, D), jnp.float32),
    mesh=_MESH,
    scratch_types=[
        pltpu.VMEM((CH,), jnp.int32),
        pltpu.VMEM((CH,), jnp.int32),
        pltpu.VMEM((CH, D), jnp.float32),
        pltpu.VMEM_SHARED((NPAD, D), jnp.float32),
        pltpu.SemaphoreType.DMA,
        pltpu.SemaphoreType.DMA,
    ],
)
def _deg_kernel(dst_hbm, ones_hbm, zeros_hbm, out_hbm,
                idv0, idv1, ones_v, acc, isem0, isem1):
    c = lax.axis_index("c")
    s = lax.axis_index("s")
    w = _wid()
    idv = (idv0, idv1)
    isem = (isem0, isem1)
    pltpu.sync_copy(zeros_hbm, acc.at[pl.ds(s * RPT, RPT)])
    pltpu.sync_copy(ones_hbm, ones_v)
    plsc.subcore_barrier()

    def fetch(k, b):
        pltpu.async_copy(dst_hbm.at[pl.ds(w * EPT + k * CH, CH)],
                         idv[b], isem[b])

    def wait_idx(b):
        pltpu.make_async_copy(dst_hbm.at[pl.ds(0, CH)], idv[b],
                              isem[b]).wait()

    fetch(0, 0)
    fetch(1, 1)

    def body(j, carry):
        for b in (0, 1):
            k = 2 * j + b
            wait_idx(b)
            pltpu.sync_copy(ones_v, acc.at[idv[b]], add=True)

            @pl.when(k <= NCHUNK - 3)
            def _():
                fetch(k + 2, b)
        return carry

    lax.fori_loop(0, (NCHUNK - 1) // 2, body, 0)
    wait_idx(0)
    pltpu.sync_copy(ones_v, acc.at[idv[0]], add=True)
    plsc.subcore_barrier()
    _writeback(acc, out_hbm, c, s)


@functools.partial(
    pl.kernel,
    out_type=jax.ShapeDtypeStruct((NC * N, D), jnp.float32),
    mesh=_MESH,
    scratch_types=[
        pltpu.VMEM((CH,), jnp.int32),
        pltpu.VMEM((CH,), jnp.int32),
        pltpu.VMEM((CH,), jnp.int32),
        pltpu.VMEM((CH,), jnp.int32),
        pltpu.VMEM((CH, D), jnp.float32),
        pltpu.VMEM((CH, D), jnp.float32),
        pltpu.VMEM_SHARED((NPAD, D), jnp.float32),
        pltpu.SemaphoreType.DMA,
        pltpu.SemaphoreType.DMA,
        pltpu.SemaphoreType.DMA,
        pltpu.SemaphoreType.DMA,
    ],
)
def _scat_kernel(src_hbm, dst_hbm, g_hbm, zeros_hbm, out_hbm,
                 isv0, isv1, idv0, idv1, rows0, rows1, acc,
                 isem0, isem1, gsem0, gsem1):
    c = lax.axis_index("c")
    s = lax.axis_index("s")
    w = _wid()
    isv = (isv0, isv1)
    idv = (idv0, idv1)
    rows = (rows0, rows1)
    isem = (isem0, isem1)
    gsem = (gsem0, gsem1)
    pltpu.sync_copy(zeros_hbm, acc.at[pl.ds(s * RPT, RPT)])
    plsc.subcore_barrier()

    def fetch(k, b):
        base = w * EPT + k * CH
        pltpu.async_copy(src_hbm.at[pl.ds(base, CH)], isv[b], isem[b])
        pltpu.async_copy(dst_hbm.at[pl.ds(base, CH)], idv[b], isem[b])

    def wait_idx(b):
        pltpu.make_async_copy(src_hbm.at[pl.ds(0, CH)], isv[b],
                              isem[b]).wait()
        pltpu.make_async_copy(dst_hbm.at[pl.ds(0, CH)], idv[b],
                              isem[b]).wait()

    def issue_gather(b):
        pltpu.async_copy(g_hbm.at[isv[b]], rows[b], gsem[b])

    def wait_gather(b):
        pltpu.make_async_copy(g_hbm.at[pl.ds(0, CH)], rows[b],
                              gsem[b]).wait()

    fetch(0, 0)
    fetch(1, 1)
    wait_idx(0)
    issue_gather(0)

    def body(j, carry):
        for b in (0, 1):
            k = 2 * j + b
            nb = 1 - b
            wait_idx(nb)
            issue_gather(nb)
            wait_gather(b)
            pltpu.sync_copy(rows[b], acc.at[idv[b]], add=True)

            @pl.when(k <= NCHUNK - 3)
            def _():
                fetch(k + 2, b)
        return carry

    lax.fori_loop(0, (NCHUNK - 1) // 2, body, 0)
    wait_gather(0)
    pltpu.sync_copy(rows[0], acc.at[idv[0]], add=True)
    plsc.subcore_barrier()
    _writeback(acc, out_hbm, c, s)


_UR = 8


@functools.partial(
    pl.kernel,
    out_type=jax.ShapeDtypeStruct((E, L), jnp.float32),
    mesh=_MESH,
    scratch_types=[
        pltpu.VMEM((CH,), jnp.int32),
        pltpu.VMEM((CH,), jnp.int32),
        pltpu.VMEM((CH,), jnp.int32),
        pltpu.VMEM((CH,), jnp.int32),
        pltpu.VMEM((CH, D), jnp.float32),
        pltpu.VMEM((CH, D), jnp.float32),
        pltpu.VMEM((CH, D), jnp.float32),
        pltpu.VMEM((CH, D), jnp.float32),
        pltpu.VMEM((CH, L), jnp.float32),
        pltpu.VMEM((CH, L), jnp.float32),
        pltpu.VMEM((DG, L), jnp.float32),
        pltpu.SemaphoreType.DMA,
        pltpu.SemaphoreType.DMA,
        pltpu.SemaphoreType.DMA,
        pltpu.SemaphoreType.DMA,
        pltpu.SemaphoreType.DMA,
        pltpu.SemaphoreType.DMA,
        pltpu.SemaphoreType.DMA,
        pltpu.SemaphoreType.DMA,
    ],
)
def _edge_kernel(src_hbm, dst_hbm, a_hbm, b_hbm, wm2_hbm, out_hbm,
                 isv0, isv1, idv0, idv1, ra0, ra1, rb0, rb1, tb0, tb1, wv,
                 isem0, isem1, ga0, ga1, gb0, gb1, ws0, ws1):
    w = _wid()
    isv = (isv0, isv1)
    idv = (idv0, idv1)
    ra = (ra0, ra1)
    rb = (rb0, rb1)
    tb = (tb0, tb1)
    isem = (isem0, isem1)
    ga = (ga0, ga1)
    gb = (gb0, gb1)
    ws = (ws0, ws1)
    pltpu.sync_copy(wm2_hbm, wv)

    def fetch(k, b):
        base = w * EPT + k * CH
        pltpu.async_copy(src_hbm.at[pl.ds(base, CH)], isv[b], isem[b])
        pltpu.async_copy(dst_hbm.at[pl.ds(base, CH)], idv[b], isem[b])

    def wait_idx(b):
        pltpu.make_async_copy(src_hbm.at[pl.ds(0, CH)], isv[b],
                              isem[b]).wait()
        pltpu.make_async_copy(dst_hbm.at[pl.ds(0, CH)], idv[b],
                              isem[b]).wait()

    def issue_gathers(b):
        pltpu.async_copy(a_hbm.at[isv[b]], ra[b], ga[b])
        pltpu.async_copy(b_hbm.at[idv[b]], rb[b], gb[b])

    def wait_gathers(b):
        pltpu.make_async_copy(a_hbm.at[pl.ds(0, CH)], ra[b], ga[b]).wait()
        pltpu.make_async_copy(b_hbm.at[pl.ds(0, CH)], rb[b], gb[b]).wait()

    def wait_wb(b):
        pltpu.make_async_copy(tb[b], out_hbm.at[pl.ds(0, CH)], ws[b]).wait()

    def compute(b):
        def rbody(rr, rc):
            for u in range(_UR):
                r = rr * _UR + u
                t0 = jnp.zeros((L,), jnp.float32)
                t1 = jnp.zeros((L,), jnp.float32)
                for cg in range(0, DG, 2):
                    av0 = ra[b][r, pl.ds(cg * L, L)]
                    bv0 = rb[b][r, pl.ds(cg * L, L)]
                    av1 = ra[b][r, pl.ds((cg + 1) * L, L)]
                    bv1 = rb[b][r, pl.ds((cg + 1) * L, L)]
                    t0 = t0 + jnp.maximum(av0 + bv0, 0.0) * wv[cg]
                    t1 = t1 + jnp.maximum(av1 + bv1, 0.0) * wv[cg + 1]
                tb[b][r] = t0 + t1
            return rc

        lax.fori_loop(0, CH // _UR, rbody, 0)

    fetch(0, 0)
    fetch(1, 1)
    wait_idx(0)
    issue_gathers(0)

    def body(j, carry):
        for b in (0, 1):
            k = 2 * j + b
            nb = 1 - b
            wait_idx(nb)
            issue_gathers(nb)
            wait_gathers(b)

            @pl.when(k >= 2)
            def _():
                wait_wb(b)

            compute(b)
            pltpu.async_copy(tb[b], out_hbm.at[pl.ds(w * EPT + k * CH, CH)],
                             ws[b])

            @pl.when(k <= NCHUNK - 3)
            def _():
                fetch(k + 2, b)
        return carry

    lax.fori_loop(0, (NCHUNK - 1) // 2, body, 0)
    wait_gathers(0)
    wait_wb(0)
    compute(0)
    pltpu.sync_copy(tb[0], out_hbm.at[pl.ds(w * EPT + (NCHUNK - 1) * CH, CH)])
    wait_wb(1)


_RB = 1000
_NB = N // _RB


def _tc1_body(dp0, dp1, x, w1, g_out, dinv_out):
    deg = dp0[...] + dp1[...] + 1.0
    dinv = lax.rsqrt(deg)
    h = jnp.dot(x[...], w1[...], preferred_element_type=jnp.float32)
    g_out[...] = dinv * h
    dinv_out[...] = dinv


def _tc2_body(g1, s0, s1, dinv, b1, w2, g2_out):
    out1 = dinv[...] * (s0[...] + s1[...] + g1[...]) + b1[...]
    h2 = jnp.maximum(out1, 0.0)
    g2_out[...] = dinv[...] * jnp.dot(h2, w2[...],
                                      preferred_element_type=jnp.float32)


def _tc3_body(g2, s0, s1, dinv, b2, wtop, wbot, bm1, a_out, b_out):
    out2 = dinv[...] * (s0[...] + s1[...] + g2[...]) + b2[...]
    h = jnp.maximum(out2, 0.0)
    a_out[...] = jnp.dot(h, wtop[...], preferred_element_type=jnp.float32) + bm1[...]
    b_out[...] = jnp.dot(h, wbot[...], preferred_element_type=jnp.float32)


_EB = 8000


def _tc4_body(t, bm2, out):
    s = jnp.sum(t[...], axis=1, keepdims=True) + bm2[0, 0]
    out[...] = jax.nn.sigmoid(s)


def _row_spec(shape):
    return pl.BlockSpec(shape, lambda i: (i, 0))


def _half2_spec(shape):
    return pl.BlockSpec(shape, lambda i: (i + _NB, 0))


def _rep_spec(shape):
    return pl.BlockSpec(shape, lambda i: (0, 0))


def kernel(x, edge_index, W1, b1, W2, b2, Wm1, bm1, Wm2, bm2):
    ei = edge_index.astype(jnp.int32)
    src = ei[0]
    dst = ei[1]

    ones_d = jnp.ones((CH, D), jnp.float32)
    zeros_d = jnp.zeros((RPT, D), jnp.float32)

    degp = _deg_kernel(dst, ones_d, zeros_d)

    grid = (_NB,)
    g1, dinvb = pl.pallas_call(
        _tc1_body,
        grid=grid,
        in_specs=[_row_spec((_RB, D)), _half2_spec((_RB, D)),
                  _row_spec((_RB, D)), _rep_spec((D, D))],
        out_specs=[_row_spec((_RB, D)), _row_spec((_RB, D))],
        out_shape=[jax.ShapeDtypeStruct((N, D), jnp.float32),
                   jax.ShapeDtypeStruct((N, D), jnp.float32)],
    )(degp, degp, x, W1)

    sc1 = _scat_kernel(src, dst, g1, zeros_d)

    g2 = pl.pallas_call(
        _tc2_body,
        grid=grid,
        in_specs=[_row_spec((_RB, D)), _row_spec((_RB, D)), _half2_spec((_RB, D)),
                  _row_spec((_RB, D)), _rep_spec((1, D)), _rep_spec((D, D))],
        out_specs=_row_spec((_RB, D)),
        out_shape=jax.ShapeDtypeStruct((N, D), jnp.float32),
    )(g1, sc1, sc1, dinvb, b1.reshape(1, D), W2)

    sc2 = _scat_kernel(src, dst, g2, zeros_d)

    a_mat, b_mat = pl.pallas_call(
        _tc3_body,
        grid=grid,
        in_specs=[_row_spec((_RB, D)), _row_spec((_RB, D)), _half2_spec((_RB, D)),
                  _row_spec((_RB, D)), _rep_spec((1, D)), _rep_spec((D, D)),
                  _rep_spec((D, D)), _rep_spec((1, D))],
        out_specs=[_row_spec((_RB, D)), _row_spec((_RB, D))],
        out_shape=[jax.ShapeDtypeStruct((N, D), jnp.float32),
                   jax.ShapeDtypeStruct((N, D), jnp.float32)],
    )(g2, sc2, sc2, dinvb, b2.reshape(1, D),
      Wm1[:D], Wm1[D:], bm1.reshape(1, D))

    t_part = _edge_kernel(src, dst, a_mat, b_mat, Wm2.reshape(DG, L))

    pred = pl.pallas_call(
        _tc4_body,
        grid=(E // _EB,),
        in_specs=[_row_spec((_EB, L)), _rep_spec((1, 1))],
        out_specs=_row_spec((_EB, 1)),
        out_shape=jax.ShapeDtypeStruct((E, 1), jnp.float32),
    )(t_part, bm2.reshape(1, 1))

    return pred.reshape(E)


# speedup vs baseline: 1.0071x; 1.0071x over previous
import functools

import jax
import jax.numpy as jnp
from jax import lax
from jax.experimental import pallas as pl
from jax.experimental.pallas import tpu as pltpu
from jax.experimental.pallas import tpu_sc as plsc

NC = 2
NS = 16
L = 16
NW = NC * NS

N = 10000
NPAD = 10240
D = 128
E = 320000
EPT = E // NW
CH = 80
NCHUNK = EPT // CH
RPT = NPAD // NS
LRPT = N - (NS - 1) * RPT
DG = D // L

_MESH = plsc.VectorSubcoreMesh(
    core_axis_name="c", subcore_axis_name="s", num_cores=NC, num_subcores=NS)


def _wid():
    return lax.axis_index("c") * NS + lax.axis_index("s")


def _writeback(acc, out_hbm, c, s):
    @pl.when(s < NS - 1)
    def _():
        pltpu.sync_copy(acc.at[pl.ds(s * RPT, RPT)],
                        out_hbm.at[pl.ds(c * N + s * RPT, RPT)])

    @pl.when(s == NS - 1)
    def _():
        pltpu.sync_copy(acc.at[pl.ds((NS - 1) * RPT, LRPT)],
                        out_hbm.at[pl.ds(c * N + (NS - 1) * RPT, LRPT)])


@functools.partial(
    pl.kernel,
    out_type=jax.ShapeDtypeStruct((NC * N, D), jnp.float32),
    mesh=_MESH,
    scratch_types=[
        pltpu.VMEM((CH,), jnp.int32),
        pltpu.VMEM((CH,), jnp.int32),
        pltpu.VMEM((CH, D), jnp.float32),
        pltpu.VMEM_SHARED((NPAD, D), jnp.float32),
        pltpu.SemaphoreType.DMA,
        pltpu.SemaphoreType.DMA,
    ],
)
def _deg_kernel(dst_hbm, ones_hbm, zeros_hbm, out_hbm,
                idv0, idv1, ones_v, acc, isem0, isem1):
    c = lax.axis_index("c")
    s = lax.axis_index("s")
    w = _wid()
    idv = (idv0, idv1)
    isem = (isem0, isem1)
    pltpu.sync_copy(zeros_hbm, acc.at[pl.ds(s * RPT, RPT)])
    pltpu.sync_copy(ones_hbm, ones_v)
    plsc.subcore_barrier()

    def fetch(k, b):
        pltpu.async_copy(dst_hbm.at[pl.ds(w * EPT + k * CH, CH)],
                         idv[b], isem[b])

    def wait_idx(b):
        pltpu.make_async_copy(dst_hbm.at[pl.ds(0, CH)], idv[b],
                              isem[b]).wait()

    fetch(0, 0)
    fetch(1, 1)

    def body(j, carry):
        for b in (0, 1):
            k = 2 * j + b
            wait_idx(b)
            pltpu.sync_copy(ones_v, acc.at[idv[b]], add=True)

            @pl.when(k <= NCHUNK - 3)
            def _():
                fetch(k + 2, b)
        return carry

    lax.fori_loop(0, (NCHUNK - 1) // 2, body, 0)
    wait_idx(0)
    pltpu.sync_copy(ones_v, acc.at[idv[0]], add=True)
    plsc.subcore_barrier()
    _writeback(acc, out_hbm, c, s)


@functools.partial(
    pl.kernel,
    out_type=jax.ShapeDtypeStruct((NC * N, D), jnp.float32),
    mesh=_MESH,
    scratch_types=[
        pltpu.VMEM((CH,), jnp.int32),
        pltpu.VMEM((CH,), jnp.int32),
        pltpu.VMEM((CH,), jnp.int32),
        pltpu.VMEM((CH,), jnp.int32),
        pltpu.VMEM((CH, D), jnp.float32),
        pltpu.VMEM((CH, D), jnp.float32),
        pltpu.VMEM_SHARED((NPAD, D), jnp.float32),
        pltpu.SemaphoreType.DMA,
        pltpu.SemaphoreType.DMA,
        pltpu.SemaphoreType.DMA,
        pltpu.SemaphoreType.DMA,
    ],
)
def _scat_kernel(src_hbm, dst_hbm, g_hbm, zeros_hbm, out_hbm,
                 isv0, isv1, idv0, idv1, rows0, rows1, acc,
                 isem0, isem1, gsem0, gsem1):
    c = lax.axis_index("c")
    s = lax.axis_index("s")
    w = _wid()
    isv = (isv0, isv1)
    idv = (idv0, idv1)
    rows = (rows0, rows1)
    isem = (isem0, isem1)
    gsem = (gsem0, gsem1)
    pltpu.sync_copy(zeros_hbm, acc.at[pl.ds(s * RPT, RPT)])
    plsc.subcore_barrier()

    def fetch(k, b):
        base = w * EPT + k * CH
        pltpu.async_copy(src_hbm.at[pl.ds(base, CH)], isv[b], isem[b])
        pltpu.async_copy(dst_hbm.at[pl.ds(base, CH)], idv[b], isem[b])

    def wait_idx(b):
        pltpu.make_async_copy(src_hbm.at[pl.ds(0, CH)], isv[b],
                              isem[b]).wait()
        pltpu.make_async_copy(dst_hbm.at[pl.ds(0, CH)], idv[b],
                              isem[b]).wait()

    def issue_gather(b):
        pltpu.async_copy(g_hbm.at[isv[b]], rows[b], gsem[b])

    def wait_gather(b):
        pltpu.make_async_copy(g_hbm.at[pl.ds(0, CH)], rows[b],
                              gsem[b]).wait()

    fetch(0, 0)
    fetch(1, 1)
    wait_idx(0)
    issue_gather(0)

    def body(j, carry):
        for b in (0, 1):
            k = 2 * j + b
            nb = 1 - b
            wait_idx(nb)
            issue_gather(nb)
            wait_gather(b)
            pltpu.sync_copy(rows[b], acc.at[idv[b]], add=True)

            @pl.when(k <= NCHUNK - 3)
            def _():
                fetch(k + 2, b)
        return carry

    lax.fori_loop(0, (NCHUNK - 1) // 2, body, 0)
    wait_gather(0)
    pltpu.sync_copy(rows[0], acc.at[idv[0]], add=True)
    plsc.subcore_barrier()
    _writeback(acc, out_hbm, c, s)


_UR = 4


@functools.partial(
    pl.kernel,
    out_type=jax.ShapeDtypeStruct((E, L), jnp.float32),
    mesh=_MESH,
    scratch_types=(
        [pltpu.VMEM((CH,), jnp.int32)] * 6
        + [pltpu.VMEM((CH, D), jnp.float32)] * 6
        + [pltpu.VMEM((CH, L), jnp.float32)] * 3
        + [pltpu.VMEM((DG, L), jnp.float32)]
        + [pltpu.SemaphoreType.DMA] * 12
    ),
)
def _edge_kernel(src_hbm, dst_hbm, a_hbm, b_hbm, wm2_hbm, out_hbm,
                 isv0, isv1, isv2, idv0, idv1, idv2,
                 ra0, ra1, ra2, rb0, rb1, rb2, tb0, tb1, tb2, wv,
                 isem0, isem1, isem2, ga0, ga1, ga2, gb0, gb1, gb2,
                 ws0, ws1, ws2):
    w = _wid()
    isv = (isv0, isv1, isv2)
    idv = (idv0, idv1, idv2)
    ra = (ra0, ra1, ra2)
    rb = (rb0, rb1, rb2)
    tb = (tb0, tb1, tb2)
    isem = (isem0, isem1, isem2)
    ga = (ga0, ga1, ga2)
    gb = (gb0, gb1, gb2)
    ws = (ws0, ws1, ws2)
    pltpu.sync_copy(wm2_hbm, wv)

    def fetch(k, b):
        base = w * EPT + k * CH
        pltpu.async_copy(src_hbm.at[pl.ds(base, CH)], isv[b], isem[b])
        pltpu.async_copy(dst_hbm.at[pl.ds(base, CH)], idv[b], isem[b])

    def wait_idx(b):
        pltpu.make_async_copy(src_hbm.at[pl.ds(0, CH)], isv[b],
                              isem[b]).wait()
        pltpu.make_async_copy(dst_hbm.at[pl.ds(0, CH)], idv[b],
                              isem[b]).wait()

    def issue_gathers(b):
        pltpu.async_copy(a_hbm.at[isv[b]], ra[b], ga[b])
        pltpu.async_copy(b_hbm.at[idv[b]], rb[b], gb[b])

    def wait_gathers(b):
        pltpu.make_async_copy(a_hbm.at[pl.ds(0, CH)], ra[b], ga[b]).wait()
        pltpu.make_async_copy(b_hbm.at[pl.ds(0, CH)], rb[b], gb[b]).wait()

    def wait_wb(b):
        pltpu.make_async_copy(tb[b], out_hbm.at[pl.ds(0, CH)], ws[b]).wait()

    def compute(b):
        def rbody(rr, rc):
            for u in range(_UR):
                r = rr * _UR + u
                t = jnp.zeros((L,), jnp.float32)
                for cg in range(DG):
                    av = ra[b][r, pl.ds(cg * L, L)]
                    bv = rb[b][r, pl.ds(cg * L, L)]
                    t = t + jnp.maximum(av + bv, 0.0) * wv[cg]
                tb[b][r] = t
            return rc

        lax.fori_loop(0, CH // _UR, rbody, 0)

    fetch(0, 0)
    fetch(1, 1)
    fetch(2, 2)
    wait_idx(0)
    issue_gathers(0)
    wait_idx(1)
    issue_gathers(1)

    def step(k, b):
        nb = (b + 2) % 3
        wait_idx(nb)
        issue_gathers(nb)
        wait_gathers(b)

        @pl.when(k >= 3)
        def _():
            wait_wb(b)

        compute(b)
        pltpu.async_copy(tb[b], out_hbm.at[pl.ds(w * EPT + k * CH, CH)],
                         ws[b])

        @pl.when(k <= NCHUNK - 4)
        def _():
            fetch(k + 3, b)

    def body(j, carry):
        for b in (0, 1, 2):
            step(3 * j + b, b)
        return carry

    lax.fori_loop(0, (NCHUNK - 2) // 3, body, 0)
    for k in (NCHUNK - 2, NCHUNK - 1):
        b = k % 3
        wait_gathers(b)
        wait_wb(b)
        compute(b)
        pltpu.async_copy(tb[b], out_hbm.at[pl.ds(w * EPT + k * CH, CH)],
                         ws[b])
    wait_wb((NCHUNK - 3) % 3)
    wait_wb((NCHUNK - 2) % 3)
    wait_wb((NCHUNK - 1) % 3)


_RB = 1000
_NB = N // _RB


def _tc1_body(dp0, dp1, x, w1, g_out, dinv_out):
    deg = dp0[...] + dp1[...] + 1.0
    dinv = lax.rsqrt(deg)
    h = jnp.dot(x[...], w1[...], preferred_element_type=jnp.float32)
    g_out[...] = dinv * h
    dinv_out[...] = dinv


def _tc2_body(g1, s0, s1, dinv, b1, w2, g2_out):
    out1 = dinv[...] * (s0[...] + s1[...] + g1[...]) + b1[...]
    h2 = jnp.maximum(out1, 0.0)
    g2_out[...] = dinv[...] * jnp.dot(h2, w2[...],
                                      preferred_element_type=jnp.float32)


def _tc3_body(g2, s0, s1, dinv, b2, wtop, wbot, bm1, a_out, b_out):
    out2 = dinv[...] * (s0[...] + s1[...] + g2[...]) + b2[...]
    h = jnp.maximum(out2, 0.0)
    a_out[...] = jnp.dot(h, wtop[...], preferred_element_type=jnp.float32) + bm1[...]
    b_out[...] = jnp.dot(h, wbot[...], preferred_element_type=jnp.float32)


_EB = 8000


def _tc4_body(t, bm2, out):
    s = jnp.sum(t[...], axis=1, keepdims=True) + bm2[0, 0]
    out[...] = jax.nn.sigmoid(s)


def _row_spec(shape):
    return pl.BlockSpec(shape, lambda i: (i, 0))


def _half2_spec(shape):
    return pl.BlockSpec(shape, lambda i: (i + _NB, 0))


def _rep_spec(shape):
    return pl.BlockSpec(shape, lambda i: (0, 0))


def kernel(x, edge_index, W1, b1, W2, b2, Wm1, bm1, Wm2, bm2):
    ei = edge_index.astype(jnp.int32)
    src = ei[0]
    dst = ei[1]

    ones_d = jnp.ones((CH, D), jnp.float32)
    zeros_d = jnp.zeros((RPT, D), jnp.float32)

    degp = _deg_kernel(dst, ones_d, zeros_d)

    grid = (_NB,)
    g1, dinvb = pl.pallas_call(
        _tc1_body,
        grid=grid,
        in_specs=[_row_spec((_RB, D)), _half2_spec((_RB, D)),
                  _row_spec((_RB, D)), _rep_spec((D, D))],
        out_specs=[_row_spec((_RB, D)), _row_spec((_RB, D))],
        out_shape=[jax.ShapeDtypeStruct((N, D), jnp.float32),
                   jax.ShapeDtypeStruct((N, D), jnp.float32)],
    )(degp, degp, x, W1)

    sc1 = _scat_kernel(src, dst, g1, zeros_d)

    g2 = pl.pallas_call(
        _tc2_body,
        grid=grid,
        in_specs=[_row_spec((_RB, D)), _row_spec((_RB, D)), _half2_spec((_RB, D)),
                  _row_spec((_RB, D)), _rep_spec((1, D)), _rep_spec((D, D))],
        out_specs=_row_spec((_RB, D)),
        out_shape=jax.ShapeDtypeStruct((N, D), jnp.float32),
    )(g1, sc1, sc1, dinvb, b1.reshape(1, D), W2)

    sc2 = _scat_kernel(src, dst, g2, zeros_d)

    a_mat, b_mat = pl.pallas_call(
        _tc3_body,
        grid=grid,
        in_specs=[_row_spec((_RB, D)), _row_spec((_RB, D)), _half2_spec((_RB, D)),
                  _row_spec((_RB, D)), _rep_spec((1, D)), _rep_spec((D, D)),
                  _rep_spec((D, D)), _rep_spec((1, D))],
        out_specs=[_row_spec((_RB, D)), _row_spec((_RB, D))],
        out_shape=[jax.ShapeDtypeStruct((N, D), jnp.float32),
                   jax.ShapeDtypeStruct((N, D), jnp.float32)],
    )(g2, sc2, sc2, dinvb, b2.reshape(1, D),
      Wm1[:D], Wm1[D:], bm1.reshape(1, D))

    t_part = _edge_kernel(src, dst, a_mat, b_mat, Wm2.reshape(DG, L))

    pred = pl.pallas_call(
        _tc4_body,
        grid=(E // _EB,),
        in_specs=[_row_spec((_EB, L)), _rep_spec((1, 1))],
        out_specs=_row_spec((_EB, 1)),
        out_shape=jax.ShapeDtypeStruct((E, 1), jnp.float32),
    )(t_part, bm2.reshape(1, 1))

    return pred.reshape(E)


# speedup vs baseline: 1.0744x; 1.0668x over previous
import functools

import jax
import jax.numpy as jnp
from jax import lax
from jax.experimental import pallas as pl
from jax.experimental.pallas import tpu as pltpu
from jax.experimental.pallas import tpu_sc as plsc

NC = 2
NS = 16
L = 16
NW = NC * NS

N = 10000
NPAD = 10240
D = 128
E = 320000
EPT = E // NW
CH = 80
NCHUNK = EPT // CH
RPT = NPAD // NS
LRPT = N - (NS - 1) * RPT
DG = D // L

_MESH = plsc.VectorSubcoreMesh(
    core_axis_name="c", subcore_axis_name="s", num_cores=NC, num_subcores=NS)


def _wid():
    return lax.axis_index("c") * NS + lax.axis_index("s")


def _writeback(acc, out_hbm, c, s):
    @pl.when(s < NS - 1)
    def _():
        pltpu.sync_copy(acc.at[pl.ds(s * RPT, RPT)],
                        out_hbm.at[pl.ds(c * N + s * RPT, RPT)])

    @pl.when(s == NS - 1)
    def _():
        pltpu.sync_copy(acc.at[pl.ds((NS - 1) * RPT, LRPT)],
                        out_hbm.at[pl.ds(c * N + (NS - 1) * RPT, LRPT)])


@functools.partial(
    pl.kernel,
    out_type=jax.ShapeDtypeStruct((NC * N, D), jnp.float32),
    mesh=_MESH,
    scratch_types=[
        pltpu.VMEM((CH,), jnp.int32),
        pltpu.VMEM((CH,), jnp.int32),
        pltpu.VMEM((CH, D), jnp.float32),
        pltpu.VMEM_SHARED((NPAD, D), jnp.float32),
        pltpu.SemaphoreType.DMA,
        pltpu.SemaphoreType.DMA,
    ],
)
def _deg_kernel(dst_hbm, ones_hbm, zeros_hbm, out_hbm,
                idv0, idv1, ones_v, acc, isem0, isem1):
    c = lax.axis_index("c")
    s = lax.axis_index("s")
    w = _wid()
    idv = (idv0, idv1)
    isem = (isem0, isem1)
    pltpu.sync_copy(zeros_hbm, acc.at[pl.ds(s * RPT, RPT)])
    pltpu.sync_copy(ones_hbm, ones_v)
    plsc.subcore_barrier()

    def fetch(k, b):
        pltpu.async_copy(dst_hbm.at[pl.ds(w * EPT + k * CH, CH)],
                         idv[b], isem[b])

    def wait_idx(b):
        pltpu.make_async_copy(dst_hbm.at[pl.ds(0, CH)], idv[b],
                              isem[b]).wait()

    fetch(0, 0)
    fetch(1, 1)

    def body(j, carry):
        for b in (0, 1):
            k = 2 * j + b
            wait_idx(b)
            pltpu.sync_copy(ones_v, acc.at[idv[b]], add=True)

            @pl.when(k <= NCHUNK - 3)
            def _():
                fetch(k + 2, b)
        return carry

    lax.fori_loop(0, (NCHUNK - 1) // 2, body, 0)
    wait_idx(0)
    pltpu.sync_copy(ones_v, acc.at[idv[0]], add=True)
    plsc.subcore_barrier()
    _writeback(acc, out_hbm, c, s)


@functools.partial(
    pl.kernel,
    out_type=jax.ShapeDtypeStruct((NC * N, D), jnp.float32),
    mesh=_MESH,
    scratch_types=[
        pltpu.VMEM((CH,), jnp.int32),
        pltpu.VMEM((CH,), jnp.int32),
        pltpu.VMEM((CH,), jnp.int32),
        pltpu.VMEM((CH,), jnp.int32),
        pltpu.VMEM((CH, D), jnp.float32),
        pltpu.VMEM((CH, D), jnp.float32),
        pltpu.VMEM_SHARED((NPAD, D), jnp.float32),
        pltpu.SemaphoreType.DMA,
        pltpu.SemaphoreType.DMA,
        pltpu.SemaphoreType.DMA,
        pltpu.SemaphoreType.DMA,
    ],
)
def _scat_kernel(src_hbm, dst_hbm, g_hbm, zeros_hbm, out_hbm,
                 isv0, isv1, idv0, idv1, rows0, rows1, acc,
                 isem0, isem1, gsem0, gsem1):
    c = lax.axis_index("c")
    s = lax.axis_index("s")
    w = _wid()
    isv = (isv0, isv1)
    idv = (idv0, idv1)
    rows = (rows0, rows1)
    isem = (isem0, isem1)
    gsem = (gsem0, gsem1)
    pltpu.sync_copy(zeros_hbm, acc.at[pl.ds(s * RPT, RPT)])
    plsc.subcore_barrier()

    def fetch(k, b):
        base = w * EPT + k * CH
        pltpu.async_copy(src_hbm.at[pl.ds(base, CH)], isv[b], isem[b])
        pltpu.async_copy(dst_hbm.at[pl.ds(base, CH)], idv[b], isem[b])

    def wait_idx(b):
        pltpu.make_async_copy(src_hbm.at[pl.ds(0, CH)], isv[b],
                              isem[b]).wait()
        pltpu.make_async_copy(dst_hbm.at[pl.ds(0, CH)], idv[b],
                              isem[b]).wait()

    def issue_gather(b):
        pltpu.async_copy(g_hbm.at[isv[b]], rows[b], gsem[b])

    def wait_gather(b):
        pltpu.make_async_copy(g_hbm.at[pl.ds(0, CH)], rows[b],
                              gsem[b]).wait()

    fetch(0, 0)
    fetch(1, 1)
    wait_idx(0)
    issue_gather(0)

    def body(j, carry):
        for b in (0, 1):
            k = 2 * j + b
            nb = 1 - b
            wait_idx(nb)
            issue_gather(nb)
            wait_gather(b)
            pltpu.sync_copy(rows[b], acc.at[idv[b]], add=True)

            @pl.when(k <= NCHUNK - 3)
            def _():
                fetch(k + 2, b)
        return carry

    lax.fori_loop(0, (NCHUNK - 1) // 2, body, 0)
    wait_gather(0)
    pltpu.sync_copy(rows[0], acc.at[idv[0]], add=True)
    plsc.subcore_barrier()
    _writeback(acc, out_hbm, c, s)


_UR = 4


@functools.partial(
    pl.kernel,
    out_type=jax.ShapeDtypeStruct((E, L), jnp.float32),
    mesh=_MESH,
    scratch_types=(
        [pltpu.VMEM((CH,), jnp.int32)] * 6
        + [pltpu.VMEM((CH, D), jnp.float32)] * 6
        + [pltpu.VMEM((CH, L), jnp.float32)] * 3
        + [pltpu.VMEM((DG, L), jnp.float32)]
        + [pltpu.SemaphoreType.DMA] * 12
    ),
)
def _edge_kernel(src_hbm, dst_hbm, a_hbm, b_hbm, wm2_hbm, out_hbm,
                 isv0, isv1, isv2, idv0, idv1, idv2,
                 ra0, ra1, ra2, rb0, rb1, rb2, tb0, tb1, tb2, wv,
                 isem0, isem1, isem2, ga0, ga1, ga2, gb0, gb1, gb2,
                 ws0, ws1, ws2):
    w = _wid()
    isv = (isv0, isv1, isv2)
    idv = (idv0, idv1, idv2)
    ra = (ra0, ra1, ra2)
    rb = (rb0, rb1, rb2)
    tb = (tb0, tb1, tb2)
    isem = (isem0, isem1, isem2)
    ga = (ga0, ga1, ga2)
    gb = (gb0, gb1, gb2)
    ws = (ws0, ws1, ws2)
    pltpu.sync_copy(wm2_hbm, wv)
    wvs = tuple(wv[cg] for cg in range(DG))

    def fetch(k, b):
        base = w * EPT + k * CH
        pltpu.async_copy(src_hbm.at[pl.ds(base, CH)], isv[b], isem[b])
        pltpu.async_copy(dst_hbm.at[pl.ds(base, CH)], idv[b], isem[b])

    def wait_idx(b):
        pltpu.make_async_copy(src_hbm.at[pl.ds(0, CH)], isv[b],
                              isem[b]).wait()
        pltpu.make_async_copy(dst_hbm.at[pl.ds(0, CH)], idv[b],
                              isem[b]).wait()

    def issue_gathers(b):
        pltpu.async_copy(a_hbm.at[isv[b]], ra[b], ga[b])
        pltpu.async_copy(b_hbm.at[idv[b]], rb[b], gb[b])

    def wait_gathers(b):
        pltpu.make_async_copy(a_hbm.at[pl.ds(0, CH)], ra[b], ga[b]).wait()
        pltpu.make_async_copy(b_hbm.at[pl.ds(0, CH)], rb[b], gb[b]).wait()

    def wait_wb(b):
        pltpu.make_async_copy(tb[b], out_hbm.at[pl.ds(0, CH)], ws[b]).wait()

    def compute(b):
        def rbody(rr, rc):
            for u in range(_UR):
                r = rr * _UR + u
                t = jnp.zeros((L,), jnp.float32)
                for cg in range(DG):
                    av = ra[b][r, pl.ds(cg * L, L)]
                    bv = rb[b][r, pl.ds(cg * L, L)]
                    t = t + jnp.maximum(av + bv, 0.0) * wvs[cg]
                tb[b][r] = t
            return rc

        lax.fori_loop(0, CH // _UR, rbody, 0)

    fetch(0, 0)
    fetch(1, 1)
    fetch(2, 2)
    wait_idx(0)
    issue_gathers(0)
    wait_idx(1)
    issue_gathers(1)

    def step(k, b):
        nb = (b + 2) % 3
        wait_idx(nb)
        issue_gathers(nb)
        wait_gathers(b)

        @pl.when(k >= 3)
        def _():
            wait_wb(b)

        compute(b)
        pltpu.async_copy(tb[b], out_hbm.at[pl.ds(w * EPT + k * CH, CH)],
                         ws[b])

        @pl.when(k <= NCHUNK - 4)
        def _():
            fetch(k + 3, b)

    def body(j, carry):
        for b in (0, 1, 2):
            step(3 * j + b, b)
        return carry

    lax.fori_loop(0, (NCHUNK - 2) // 3, body, 0)
    for k in (NCHUNK - 2, NCHUNK - 1):
        b = k % 3
        wait_gathers(b)
        wait_wb(b)
        compute(b)
        pltpu.async_copy(tb[b], out_hbm.at[pl.ds(w * EPT + k * CH, CH)],
                         ws[b])
    wait_wb((NCHUNK - 3) % 3)
    wait_wb((NCHUNK - 2) % 3)
    wait_wb((NCHUNK - 1) % 3)


_RB = 1000
_NB = N // _RB


def _tc1_body(dp0, dp1, x, w1, g_out, dinv_out):
    deg = dp0[...] + dp1[...] + 1.0
    dinv = lax.rsqrt(deg)
    h = jnp.dot(x[...], w1[...], preferred_element_type=jnp.float32)
    g_out[...] = dinv * h
    dinv_out[...] = dinv


def _tc2_body(g1, s0, s1, dinv, b1, w2, g2_out):
    out1 = dinv[...] * (s0[...] + s1[...] + g1[...]) + b1[...]
    h2 = jnp.maximum(out1, 0.0)
    g2_out[...] = dinv[...] * jnp.dot(h2, w2[...],
                                      preferred_element_type=jnp.float32)


def _tc3_body(g2, s0, s1, dinv, b2, wtop, wbot, bm1, a_out, b_out):
    out2 = dinv[...] * (s0[...] + s1[...] + g2[...]) + b2[...]
    h = jnp.maximum(out2, 0.0)
    a_out[...] = jnp.dot(h, wtop[...], preferred_element_type=jnp.float32) + bm1[...]
    b_out[...] = jnp.dot(h, wbot[...], preferred_element_type=jnp.float32)


_EB = 2000


def _tc4_body(p, m, bm2s, out):
    q = jnp.dot(p[...], m[...], preferred_element_type=jnp.float32)
    out[...] = 1.0 / (1.0 + jnp.exp(-(q + bm2s[...])))


def _row_spec(shape):
    return pl.BlockSpec(shape, lambda i: (i, 0))


def _half2_spec(shape):
    return pl.BlockSpec(shape, lambda i: (i + _NB, 0))


def _rep_spec(shape):
    return pl.BlockSpec(shape, lambda i: (0, 0))


def kernel(x, edge_index, W1, b1, W2, b2, Wm1, bm1, Wm2, bm2):
    ei = edge_index.astype(jnp.int32)
    src = ei[0]
    dst = ei[1]

    ones_d = jnp.ones((CH, D), jnp.float32)
    zeros_d = jnp.zeros((RPT, D), jnp.float32)

    degp = _deg_kernel(dst, ones_d, zeros_d)

    grid = (_NB,)
    g1, dinvb = pl.pallas_call(
        _tc1_body,
        grid=grid,
        in_specs=[_row_spec((_RB, D)), _half2_spec((_RB, D)),
                  _row_spec((_RB, D)), _rep_spec((D, D))],
        out_specs=[_row_spec((_RB, D)), _row_spec((_RB, D))],
        out_shape=[jax.ShapeDtypeStruct((N, D), jnp.float32),
                   jax.ShapeDtypeStruct((N, D), jnp.float32)],
    )(degp, degp, x, W1)

    sc1 = _scat_kernel(src, dst, g1, zeros_d)

    g2 = pl.pallas_call(
        _tc2_body,
        grid=grid,
        in_specs=[_row_spec((_RB, D)), _row_spec((_RB, D)), _half2_spec((_RB, D)),
                  _row_spec((_RB, D)), _rep_spec((1, D)), _rep_spec((D, D))],
        out_specs=_row_spec((_RB, D)),
        out_shape=jax.ShapeDtypeStruct((N, D), jnp.float32),
    )(g1, sc1, sc1, dinvb, b1.reshape(1, D), W2)

    sc2 = _scat_kernel(src, dst, g2, zeros_d)

    a_mat, b_mat = pl.pallas_call(
        _tc3_body,
        grid=grid,
        in_specs=[_row_spec((_RB, D)), _row_spec((_RB, D)), _half2_spec((_RB, D)),
                  _row_spec((_RB, D)), _rep_spec((1, D)), _rep_spec((D, D)),
                  _rep_spec((D, D)), _rep_spec((1, D))],
        out_specs=[_row_spec((_RB, D)), _row_spec((_RB, D))],
        out_shape=[jax.ShapeDtypeStruct((N, D), jnp.float32),
                   jax.ShapeDtypeStruct((N, D), jnp.float32)],
    )(g2, sc2, sc2, dinvb, b2.reshape(1, D),
      Wm1[:D], Wm1[D:], bm1.reshape(1, D))

    part = _edge_kernel(src, dst, a_mat, b_mat, Wm2.reshape(DG, L))

    pp = part.reshape(E // 8, 8 * L)
    m8 = jnp.repeat(jnp.eye(8, dtype=jnp.float32), L, axis=0)
    pred8 = pl.pallas_call(
        _tc4_body,
        grid=(E // 8 // _EB,),
        in_specs=[_row_spec((_EB, 8 * L)), _rep_spec((8 * L, 8)),
                  _rep_spec((1, 8))],
        out_specs=_row_spec((_EB, 8)),
        out_shape=jax.ShapeDtypeStruct((E // 8, 8), jnp.float32),
    )(pp, m8, jnp.broadcast_to(bm2, (1, 8)).astype(jnp.float32))
    return pred8.reshape(E)
